# baseline (speedup 1.0000x reference)
import jax
import jax.numpy as jnp
from jax.experimental import pallas as pl
from jax.experimental.pallas import tpu as pltpu
from jax.experimental.pallas import tpu_sc as plsc

EMBED_DIM = 64
WINDOW = 128


def kernel(input_seqs, table):
    batch, seq = input_seqs.shape
    num_indices = batch * seq
    indices = input_seqs.reshape(1, num_indices).astype(jnp.int32)

    mesh = plsc.VectorSubcoreMesh(core_axis_name="c", subcore_axis_name="s")

    @pl.kernel(
        out_type=jax.ShapeDtypeStruct((num_indices, EMBED_DIM), table.dtype),
        mesh=mesh,
        compiler_params=pltpu.CompilerParams(use_tc_tiling_on_sc=False),
    )
    def gather_kernel(table_hbm, idx_hbm, out_hbm):
        def body(idx_vmem, out_vmem):
            pltpu.sync_copy(table_hbm.at[idx_vmem.at[0]], out_vmem)

        pltpu.emit_pipeline(
            body,
            grid=(num_indices // WINDOW,),
            in_specs=[pl.BlockSpec((1, WINDOW), index_map=lambda i: (0, i))],
            out_specs=[pl.BlockSpec((WINDOW, EMBED_DIM), index_map=lambda i: (i, 0))],
            core_axis_name=("c", "s"),
            dimension_semantics=(pltpu.PARALLEL,),
        )(idx_hbm, out_hbm)

    out = gather_kernel(table, indices)
    return out.reshape(batch, seq, EMBED_DIM)


# speedup vs baseline: 1.0290x; 1.0290x over previous
import jax
from jax import lax
import jax.numpy as jnp
from jax.experimental import pallas as pl
from jax.experimental.pallas import tpu as pltpu
from jax.experimental.pallas import tpu_sc as plsc

EMBED_DIM = 64
CHUNK = 128
NBUF = 5
NUM_CORES = 2
NUM_SUBCORES = 16
NUM_WORKERS = NUM_CORES * NUM_SUBCORES


def kernel(input_seqs, table):
    batch, seq = input_seqs.shape
    num_indices = batch * seq
    per_worker = num_indices // NUM_WORKERS
    num_chunks = per_worker // CHUNK
    num_groups = num_chunks // NBUF
    indices = input_seqs.reshape(num_indices).astype(jnp.int32)

    mesh = plsc.VectorSubcoreMesh(core_axis_name="c", subcore_axis_name="s")

    @pl.kernel(
        out_type=jax.ShapeDtypeStruct((num_indices, EMBED_DIM), table.dtype),
        mesh=mesh,
        scratch_types=[
            pltpu.VMEM((per_worker,), jnp.int32),
            pltpu.VMEM((NBUF, CHUNK, EMBED_DIM), jnp.float32),
            pltpu.SemaphoreType.DMA((NBUF,)),
            pltpu.SemaphoreType.DMA((NBUF,)),
            pltpu.SemaphoreType.DMA,
        ],
        compiler_params=pltpu.CompilerParams(use_tc_tiling_on_sc=False),
    )
    def gather_kernel(table_hbm, idx_hbm, out_hbm, idx_v, rows_v, gsem, osem, isem):
        wid = lax.axis_index("s") * NUM_CORES + lax.axis_index("c")
        base = wid * per_worker
        pltpu.async_copy(idx_hbm.at[pl.ds(base, per_worker)], idx_v, isem).wait()

        def gather(c, b):
            return pltpu.make_async_copy(
                table_hbm.at[idx_v.at[pl.ds(c * CHUNK, CHUNK)]],
                rows_v.at[b],
                gsem.at[b],
            )

        def put(c, b):
            return pltpu.make_async_copy(
                rows_v.at[b],
                out_hbm.at[pl.ds(base + c * CHUNK, CHUNK)],
                osem.at[b],
            )

        for b in range(NBUF):
            gather(b, b).start()

        @pl.loop(0, num_groups - 1)
        def _(g):
            for b in range(NBUF):
                c = g * NBUF + b
                gather(c, b).wait()
                put(c, b).start()
            for b in range(NBUF):
                c = g * NBUF + b
                put(c, b).wait()
                gather(c + NBUF, b).start()

        for b in range(NBUF):
            c = (num_groups - 1) * NBUF + b
            gather(c, b).wait()
            put(c, b).start()
        for b in range(NBUF):
            c = (num_groups - 1) * NBUF + b
            put(c, b).wait()

    out = gather_kernel(table, indices)
    return out.reshape(batch, seq, EMBED_DIM)
